# baseline (speedup 1.0000x reference)
import jax
import jax.numpy as jnp
from jax.experimental import pallas as pl
from jax.experimental.pallas import tpu as pltpu

T = 48
C = 34
WAVE = 4
HIDDEN = 64
N_EMBED = 256
BLOCKS = 4
PATCH_DIM = WAVE * C
N_TOK = T // WAVE


def _fused_body(vl_ref, x_hbm, win_hbm, bin_hbm, w1_hbm, b1_hbm, w2_hbm,
                b2_hbm, cb_hbm, out_ref,
                x_v, win_v, bin_v, w1_v, b1_v, w2_v, b2_v, cb_v,
                sx, swin, sbin, sw1, sb1, sw2, sb2, scb):
    cp_x = pltpu.make_async_copy(x_hbm, x_v, sx)
    cp_win = pltpu.make_async_copy(win_hbm, win_v, swin)
    cp_bin = pltpu.make_async_copy(bin_hbm, bin_v, sbin)
    cp_w1 = pltpu.make_async_copy(w1_hbm, w1_v, sw1)
    cp_b1 = pltpu.make_async_copy(b1_hbm, b1_v, sb1)
    cp_w2 = pltpu.make_async_copy(w2_hbm, w2_v, sw2)
    cp_b2 = pltpu.make_async_copy(b2_hbm, b2_v, sb2)
    cp_cb = pltpu.make_async_copy(cb_hbm, cb_v, scb)
    for cp in (cp_x, cp_win, cp_bin, cp_w1, cp_b1, cp_w2, cp_b2, cp_cb):
        cp.start()

    vl = vl_ref[0, 0]
    cp_x.wait()
    x = x_v[...]
    row = jax.lax.broadcasted_iota(jnp.int32, (N_TOK, PATCH_DIM), 0)
    col = jax.lax.broadcasted_iota(jnp.int32, (N_TOK, PATCH_DIM), 1)
    t = row * WAVE + col // C
    x = jnp.where(t < vl, x, 0.0)

    cp_win.wait()
    cp_bin.wait()
    z = jnp.dot(x, win_v[...], preferred_element_type=jnp.float32)
    z = z + bin_v[...]

    cp_w1.wait()
    cp_b1.wait()
    cp_w2.wait()
    cp_b2.wait()
    for i in range(BLOCKS):
        mu = z.mean(axis=-1, keepdims=True)
        var = ((z - mu) ** 2).mean(axis=-1, keepdims=True)
        h = (z - mu) / jnp.sqrt(var + 1e-5)
        h = jnp.dot(h, w1_v[i], preferred_element_type=jnp.float32) + b1_v[i][None, :]
        h = jax.nn.gelu(h)
        h = jnp.dot(h, w2_v[i], preferred_element_type=jnp.float32) + b2_v[i][None, :]
        z = z + h

    cp_cb.wait()
    cb = cb_v[...]
    z2 = jnp.sum(z * z, axis=-1, keepdims=True)
    zc = jax.lax.dot_general(z, cb, (((1,), (1,)), ((), ())),
                             preferred_element_type=jnp.float32)
    c2 = jnp.sum(cb * cb, axis=-1)
    d = z2 - 2.0 * zc + c2[None, :]

    m = jnp.min(d, axis=-1, keepdims=True)
    idx = jax.lax.broadcasted_iota(jnp.int32, (N_TOK, N_EMBED), 1)
    ids = jnp.min(jnp.where(d == m, idx, N_EMBED), axis=-1)
    out_ref[...] = jnp.broadcast_to(ids[:, None], (N_TOK, 128))


def kernel(ts, W_in, b_in, blocks_W1, blocks_b1, blocks_W2, blocks_b2,
           codebook, W_out, b_out, valid_len):
    patches = ts.reshape(N_TOK, PATCH_DIM)
    vl = jnp.asarray(valid_len, jnp.int32).reshape(1, 1)
    hbm = pl.BlockSpec(memory_space=pltpu.MemorySpace.HBM)
    out = pl.pallas_call(
        _fused_body,
        out_shape=jax.ShapeDtypeStruct((N_TOK, 128), jnp.int32),
        in_specs=[pl.BlockSpec(memory_space=pltpu.SMEM)] + [hbm] * 8,
        out_specs=pl.BlockSpec(memory_space=pltpu.VMEM),
        scratch_shapes=[
            pltpu.VMEM((N_TOK, PATCH_DIM), jnp.float32),
            pltpu.VMEM((PATCH_DIM, HIDDEN), jnp.float32),
            pltpu.VMEM((1, HIDDEN), jnp.float32),
            pltpu.VMEM((BLOCKS, HIDDEN, 4 * HIDDEN), jnp.float32),
            pltpu.VMEM((BLOCKS, 4 * HIDDEN), jnp.float32),
            pltpu.VMEM((BLOCKS, 4 * HIDDEN, HIDDEN), jnp.float32),
            pltpu.VMEM((BLOCKS, HIDDEN), jnp.float32),
            pltpu.VMEM((N_EMBED, HIDDEN), jnp.float32),
        ] + [pltpu.SemaphoreType.DMA] * 8,
    )(vl, patches, W_in, b_in.reshape(1, HIDDEN), blocks_W1, blocks_b1,
      blocks_W2, blocks_b2, codebook)
    return out[:, 0].reshape(1, N_TOK)
